# baseline (speedup 1.0000x reference)
import functools

import jax
import jax.numpy as jnp
from jax import lax
from jax.experimental import pallas as pl
from jax.experimental.pallas import tpu as pltpu
from jax.experimental.pallas import tpu_sc as plsc

_N = 10000
_E = 320000
_LAT = 64

_BN = 10000
_BE = 10000


def _tc_call(fn, n, block, ins, out_shapes):
    grid = n // block
    in_specs = []
    for a in ins:
        if a.shape[0] == n:
            in_specs.append(pl.BlockSpec((block,) + a.shape[1:],
                                         lambda i, nd=a.ndim: (i,) + (0,) * (nd - 1)))
        else:
            in_specs.append(pl.BlockSpec(a.shape,
                                         lambda i, nd=a.ndim: (0,) * nd))
    out_specs = [pl.BlockSpec((block,) + o.shape[1:],
                              lambda i, nd=o.ndim: (i,) + (0,) * (nd - 1))
                 for o in out_shapes]
    return pl.pallas_call(
        fn, grid=(grid,), in_specs=in_specs, out_specs=out_specs,
        out_shape=list(out_shapes),
    )(*ins)


def _dot(a, b):
    return jnp.dot(a, b, preferred_element_type=jnp.float32)


def _f32(shape):
    return jax.ShapeDtypeStruct(shape, jnp.float32)


def _softplus(v):
    return jnp.maximum(v, 0.0) + jnp.log(1.0 + jnp.exp(-jnp.abs(v)))


def _enc_in(x, h, maskcol, wx, wh, b, wms, wmd):
    n = x.shape[0]

    def body(x_ref, h_ref, m_ref, wx_ref, wh_ref, b_ref, wms_ref, wmd_ref,
             hid_ref, a_ref, b2_ref):
        xm = x_ref[...] * m_ref[...]
        hid = jnp.maximum(_dot(xm, wx_ref[...]) + _dot(h_ref[...], wh_ref[...])
                          + b_ref[...], 0.0)
        hid_ref[...] = hid
        a_ref[...] = _dot(hid, wms_ref[...])
        b2_ref[...] = _dot(hid, wmd_ref[...])

    return _tc_call(body, n, _BN, [x, h, maskcol, wx, wh, b, wms, wmd],
                    [_f32((n, _LAT))] * 3)


def _dec_in(z, zgp, h, wz, wzgp, wh, b, wms, wmd):
    n = z.shape[0]

    def body(z_ref, zgp_ref, h_ref, wz_ref, wzgp_ref, wh_ref, b_ref,
             wms_ref, wmd_ref, hid_ref, a_ref, b2_ref):
        hid = jnp.maximum(_dot(z_ref[...], wz_ref[...])
                          + _dot(zgp_ref[...], wzgp_ref[...])
                          + _dot(h_ref[...], wh_ref[...]) + b_ref[...], 0.0)
        hid_ref[...] = hid
        a_ref[...] = _dot(hid, wms_ref[...])
        b2_ref[...] = _dot(hid, wmd_ref[...])

    return _tc_call(body, n, _BN, [z, zgp, h, wz, wzgp, wh, b, wms, wmd],
                    [_f32((n, _LAT))] * 3)


def _ce_pair(ea, we, be, wme0, bm0, wme1, bm1):
    n = ea.shape[0]

    def body(ea_ref, we_ref, be_ref, w0_ref, b0_ref, w1_ref, b1_ref,
             c0_ref, c1_ref):
        e = jnp.maximum(_dot(ea_ref[...], we_ref[...]) + be_ref[...], 0.0)
        c0_ref[...] = _dot(e, w0_ref[...]) + b0_ref[...]
        c1_ref[...] = _dot(e, w1_ref[...]) + b1_ref[...]

    return _tc_call(body, n, _BE, [ea, we, be, wme0, bm0, wme1, bm1],
                    [_f32((n, _LAT))] * 2)


def _upd_mid(hid, p0, p1, wuh, wua, bu, wms, wmd):
    n = hid.shape[0]

    def body(hid_ref, p0_ref, p1_ref, wuh_ref, wua_ref, bu_ref,
             wms_ref, wmd_ref, o_ref, a_ref, b_ref):
        agg = p0_ref[...] + p1_ref[...]
        nh = jnp.maximum(_dot(hid_ref[...], wuh_ref[...])
                         + _dot(agg, wua_ref[...]) + bu_ref[...], 0.0)
        o_ref[...] = nh
        a_ref[...] = _dot(nh, wms_ref[...])
        b_ref[...] = _dot(nh, wmd_ref[...])

    return _tc_call(body, n, _BN, [hid, p0, p1, wuh, wua, bu, wms, wmd],
                    [_f32((n, _LAT))] * 3)


def _upd_last(hid, p0, p1, wuh, wua, bu, wout, bout):
    n = hid.shape[0]
    dout = wout.shape[1]

    def body(hid_ref, p0_ref, p1_ref, wuh_ref, wua_ref, bu_ref,
             wo_ref, bo_ref, z_ref):
        agg = p0_ref[...] + p1_ref[...]
        nh = jnp.maximum(_dot(hid_ref[...], wuh_ref[...])
                         + _dot(agg, wua_ref[...]) + bu_ref[...], 0.0)
        z_ref[...] = _dot(nh, wo_ref[...]) + bo_ref[...]

    return _tc_call(body, n, _BN, [hid, p0, p1, wuh, wua, bu, wout, bout],
                    [_f32((n, dout))])[0]


def _vae_prior(zgp, w1, b1, w2, b2):
    n = zgp.shape[0]

    def body(z_ref, w1_ref, b1_ref, w2_ref, b2_ref, loc_ref, sc_ref):
        hdn = jnp.maximum(_dot(z_ref[...], w1_ref[...]) + b1_ref[...], 0.0)
        out = _dot(hdn, w2_ref[...]) + b2_ref[...]
        loc_ref[...] = out[:, :_LAT]
        sc_ref[...] = _softplus(out[:, _LAT:]) + 1e-4

    return _tc_call(body, n, _BN, [zgp, w1, b1, w2, b2],
                    [_f32((n, _LAT))] * 2)


def _vae_inf(zg, zgp, epsv, w1a, w1b, b1, w2, b2):
    n = zg.shape[0]

    def body(zg_ref, zgp_ref, e_ref, w1a_ref, w1b_ref, b1_ref, w2_ref, b2_ref,
             loc_ref, sc_ref, z_ref):
        hdn = jnp.maximum(_dot(zg_ref[...], w1a_ref[...])
                          + _dot(zgp_ref[...], w1b_ref[...]) + b1_ref[...], 0.0)
        out = _dot(hdn, w2_ref[...]) + b2_ref[...]
        loc = out[:, :_LAT]
        sc = _softplus(out[:, _LAT:]) + 1e-4
        loc_ref[...] = loc
        sc_ref[...] = sc
        z_ref[...] = loc + sc * e_ref[...]

    return _tc_call(body, n, _BN, [zg, zgp, epsv, w1a, w1b, b1, w2, b2],
                    [_f32((n, _LAT))] * 3)


_NC = 2
_NS = 16
_NW = _NC * _NS
_EPW = _E // _NW
_SUB = 125
_NCH = _EPW // _SUB
_RB = 624
_ZR = 48
_NB = 2


def _sc_edge_call(has_ce):
    mesh = plsc.VectorSubcoreMesh(core_axis_name="c", subcore_axis_name="s")
    scratch = [
        pltpu.VMEM((_NCH, _SUB), jnp.int32),
        pltpu.VMEM((_NCH, _SUB), jnp.int32),
        pltpu.VMEM((_NB, _SUB, _LAT), jnp.float32),
        pltpu.VMEM((_NB, _SUB, _LAT), jnp.float32),
        pltpu.VMEM((_NB, _SUB, _LAT), jnp.float32),
        pltpu.VMEM((_ZR, _LAT), jnp.float32),
        pltpu.VMEM_SHARED((_N, _LAT), jnp.float32),
    ] + [pltpu.SemaphoreType.DMA] * (2 * _NB)
    if has_ce:
        scratch.insert(4, pltpu.VMEM((_NB, _SUB, _LAT), jnp.float32))
    else:
        scratch.insert(4, pltpu.VMEM((_LAT,), jnp.float32))

    def body(*refs):
        if has_ce:
            (a_hbm, b_hbm, ce_hbm, src_hbm, dst_hbm, out_hbm,
             sidx2, didx2, bufa, bufb, bufc, bufm, zb, agg) = refs[:14]
        else:
            (a_hbm, b_hbm, bias_hbm, src_hbm, dst_hbm, out_hbm,
             sidx2, didx2, bufa, bufb, bias_v, bufm, zb, agg) = refs[:14]
        fs = refs[14:14 + _NB]
        ss = refs[14 + _NB:14 + 2 * _NB]
        c = lax.axis_index("c")
        s = lax.axis_index("s")
        wid = s * _NC + c

        pltpu.sync_copy(src_hbm.at[wid], sidx2)
        pltpu.sync_copy(dst_hbm.at[wid], didx2)

        @pl.loop(0, _ZR)
        def _zero_rows(r):
            z = jnp.zeros((16,), jnp.float32)
            for kk in range(4):
                zb[r, pl.ds(kk * 16, 16)] = z

        for j in range(_RB // _ZR):
            pltpu.sync_copy(zb, agg.at[pl.ds(s * _RB + j * _ZR, _ZR), :])

        @pl.when(s == _NS - 1)
        def _zero_tail():
            pltpu.sync_copy(zb.at[pl.ds(0, _N - _NS * _RB), :],
                            agg.at[pl.ds(_NS * _RB, _N - _NS * _RB), :])
        if not has_ce:
            pltpu.sync_copy(bias_hbm, bias_v)
        plsc.subcore_barrier()

        def fire_fetch(ch, b):
            pltpu.make_async_copy(a_hbm.at[sidx2.at[ch]], bufa.at[b],
                                  fs[b]).start()
            pltpu.make_async_copy(b_hbm.at[didx2.at[ch]], bufb.at[b],
                                  fs[b]).start()
            if has_ce:
                base = wid * _EPW + ch * _SUB
                pltpu.make_async_copy(ce_hbm.at[pl.ds(base, _SUB), :],
                                      bufc.at[b], fs[b]).start()

        def wait_fetch(ch, b):
            pltpu.make_async_copy(a_hbm.at[sidx2.at[ch]], bufa.at[b],
                                  fs[b]).wait()
            pltpu.make_async_copy(b_hbm.at[didx2.at[ch]], bufb.at[b],
                                  fs[b]).wait()
            if has_ce:
                base = wid * _EPW + ch * _SUB
                pltpu.make_async_copy(ce_hbm.at[pl.ds(base, _SUB), :],
                                      bufc.at[b], fs[b]).wait()

        def wait_scatter(ch, b):
            pltpu.make_async_copy(bufm.at[b], agg.at[didx2.at[ch]],
                                  ss[b]).wait()

        for b in range(_NB - 1):
            fire_fetch(b, b)

        if not has_ce:
            bv = [bias_v[pl.ds(kk * 16, 16)] for kk in range(4)]

        @pl.loop(0, _NCH, step=_NB)
        def _outer(i0):
            for b in range(_NB):
                ch = i0 + b
                nf = ch + _NB - 1
                bf = (b + _NB - 1) % _NB

                def _refill():
                    wait_scatter(ch, bf)
                    fire_fetch(nf, bf)

                if b == 0:
                    @pl.when(i0 > 0)
                    def _():
                        _refill()

                    @pl.when(i0 == 0)
                    def _():
                        fire_fetch(nf, bf)
                else:
                    @pl.when(nf < _NCH)
                    def _():
                        _refill()

                wait_fetch(ch, b)

                if has_ce:
                    @plsc.parallel_loop(0, _SUB, unroll=4)
                    def _relu(r):
                        for kk in range(4):
                            sl = pl.ds(kk * 16, 16)
                            bufm[b, r, sl] = jnp.maximum(
                                bufa[b, r, sl] + bufb[b, r, sl]
                                + bufc[b, r, sl], 0.0)
                else:
                    @plsc.parallel_loop(0, _SUB, unroll=4)
                    def _relu(r):
                        for kk in range(4):
                            sl = pl.ds(kk * 16, 16)
                            bufm[b, r, sl] = jnp.maximum(
                                bufa[b, r, sl] + bufb[b, r, sl] + bv[kk], 0.0)

                pltpu.make_async_copy(bufm.at[b], agg.at[didx2.at[ch]],
                                      ss[b]).start(add=True)

        for b in range(_NB):
            wait_scatter(_NCH - _NB + b, b)

        plsc.subcore_barrier()
        for j in range(_RB // _ZR):
            pltpu.sync_copy(agg.at[pl.ds(s * _RB + j * _ZR, _ZR), :],
                            out_hbm.at[c, pl.ds(s * _RB + j * _ZR, _ZR), :])

        @pl.when(s == _NS - 1)
        def _write_tail():
            pltpu.sync_copy(agg.at[pl.ds(_NS * _RB, _N - _NS * _RB), :],
                            out_hbm.at[c, pl.ds(_NS * _RB, _N - _NS * _RB), :])

    return pl.kernel(
        body, mesh=mesh,
        out_type=jax.ShapeDtypeStruct((_NC, _N, _LAT), jnp.float32),
        scratch_types=scratch,
        compiler_params=pltpu.CompilerParams(use_tc_tiling_on_sc=False),
    )


def _edge_stage(a_nodes, b_nodes, ce, src, dst, dec_bias=None):
    src3 = src.reshape(_NW, _NCH, _SUB)
    dst3 = dst.reshape(_NW, _NCH, _SUB)
    if ce is not None:
        parts = _sc_edge_call(True)(a_nodes, b_nodes, ce, src3, dst3)
    else:
        parts = _sc_edge_call(False)(a_nodes, b_nodes, dec_bias.reshape(_LAT),
                                     src3, dst3)
    return parts[0], parts[1]


def _r2(v):
    return v.reshape(1, -1)


def _gnn_encode(p, x, h, maskcol, ea, src, dst):
    l0, l1 = p["layers"]
    win = p["W_in"]
    dx = win.shape[0] - h.shape[1]
    hid, a0, b0 = _enc_in(x, h, maskcol, win[:dx], win[dx:], _r2(p["b_in"]),
                          l0["W_m"][:_LAT], l0["W_m"][_LAT:2 * _LAT])
    ce0, ce1 = _ce_pair(ea, p["W_e"], _r2(p["b_e"]),
                        l0["W_m"][2 * _LAT:], _r2(l0["b_m"]),
                        l1["W_m"][2 * _LAT:], _r2(l1["b_m"]))
    q0, q1 = _edge_stage(a0, b0, ce0, src, dst)
    hid, a1, b1 = _upd_mid(hid, q0, q1, l0["W_u"][:_LAT], l0["W_u"][_LAT:],
                           _r2(l0["b_u"]), l1["W_m"][:_LAT],
                           l1["W_m"][_LAT:2 * _LAT])
    q0, q1 = _edge_stage(a1, b1, ce1, src, dst)
    return _upd_last(hid, q0, q1, l1["W_u"][:_LAT], l1["W_u"][_LAT:],
                     _r2(l1["b_u"]), p["W_out"], _r2(p["b_out"]))


def _gnn_decode(p, z, zgp, h, src, dst):
    l0, l1 = p["layers"]
    win = p["W_in"]
    e_const = jnp.maximum(p["b_e"], 0.0)
    cb0 = _r2(e_const @ l0["W_m"][2 * _LAT:] + l0["b_m"])
    cb1 = _r2(e_const @ l1["W_m"][2 * _LAT:] + l1["b_m"])
    hid, a0, b0 = _dec_in(z, zgp, h, win[:_LAT], win[_LAT:2 * _LAT],
                          win[2 * _LAT:], _r2(p["b_in"]),
                          l0["W_m"][:_LAT], l0["W_m"][_LAT:2 * _LAT])
    q0, q1 = _edge_stage(a0, b0, None, src, dst, dec_bias=cb0)
    hid, a1, b1 = _upd_mid(hid, q0, q1, l0["W_u"][:_LAT], l0["W_u"][_LAT:],
                           _r2(l0["b_u"]), l1["W_m"][:_LAT],
                           l1["W_m"][_LAT:2 * _LAT])
    q0, q1 = _edge_stage(a1, b1, None, src, dst, dec_bias=cb1)
    return _upd_last(hid, q0, q1, l1["W_u"][:_LAT], l1["W_u"][_LAT:],
                     _r2(l1["b_u"]), p["W_out"], _r2(p["b_out"]))


def kernel(x, h, edge_attr, edge_attr_partial, edge_index, partial_goal_mask,
           params, eps):
    src = edge_index[0]
    dst = edge_index[1]
    ones = jnp.ones((x.shape[0], 1), jnp.float32)
    maskcol = partial_goal_mask[:, None]

    z_goal = _gnn_encode(params["goal"], x, h, ones, edge_attr, src, dst)
    z_gp = _gnn_encode(params["partial"], x, h, maskcol, edge_attr_partial,
                       src, dst)

    pv = params["prior"]
    loc_p, scale_p = _vae_prior(z_gp, pv["W1"], _r2(pv["b1"]),
                                pv["W2"], _r2(pv["b2"]))
    iv = params["inf"]
    loc_q, scale_q, z = _vae_inf(z_goal, z_gp, eps,
                                 iv["W1"][:_LAT], iv["W1"][_LAT:], _r2(iv["b1"]),
                                 iv["W2"], _r2(iv["b2"]))

    mu = _gnn_decode(params["dec"], z, z_gp, h, src, dst)
    return mu, loc_q, scale_q, loc_p, scale_p


# speedup vs baseline: 1.3370x; 1.3370x over previous
import functools

import jax
import jax.numpy as jnp
from jax import lax
from jax.experimental import pallas as pl
from jax.experimental.pallas import tpu as pltpu
from jax.experimental.pallas import tpu_sc as plsc

_N = 10000
_E = 320000
_LAT = 64

_BN = 10000
_BE = 10000


def _tc_call(fn, n, block, ins, out_shapes):
    grid = n // block
    in_specs = []
    for a in ins:
        if a.shape[0] == n:
            in_specs.append(pl.BlockSpec((block,) + a.shape[1:],
                                         lambda i, nd=a.ndim: (i,) + (0,) * (nd - 1)))
        else:
            in_specs.append(pl.BlockSpec(a.shape,
                                         lambda i, nd=a.ndim: (0,) * nd))
    out_specs = [pl.BlockSpec((block,) + o.shape[1:],
                              lambda i, nd=o.ndim: (i,) + (0,) * (nd - 1))
                 for o in out_shapes]
    return pl.pallas_call(
        fn, grid=(grid,), in_specs=in_specs, out_specs=out_specs,
        out_shape=list(out_shapes),
    )(*ins)


def _dot(a, b):
    return jnp.dot(a, b, preferred_element_type=jnp.float32)


def _f32(shape):
    return jax.ShapeDtypeStruct(shape, jnp.float32)


def _softplus(v):
    return jnp.maximum(v, 0.0) + jnp.log(1.0 + jnp.exp(-jnp.abs(v)))


def _enc_in(x, h, maskcol, wx, wh, b, wms, wmd):
    n = x.shape[0]

    def body(x_ref, h_ref, m_ref, wx_ref, wh_ref, b_ref, wms_ref, wmd_ref,
             hid_ref, a_ref, b2_ref):
        xm = x_ref[...] * m_ref[...]
        hid = jnp.maximum(_dot(xm, wx_ref[...]) + _dot(h_ref[...], wh_ref[...])
                          + b_ref[...], 0.0)
        hid_ref[...] = hid
        a_ref[...] = _dot(hid, wms_ref[...])
        b2_ref[...] = _dot(hid, wmd_ref[...])

    return _tc_call(body, n, _BN, [x, h, maskcol, wx, wh, b, wms, wmd],
                    [_f32((n, _LAT))] * 3)


def _dec_in(z, zgp, h, wz, wzgp, wh, b, wms, wmd):
    n = z.shape[0]

    def body(z_ref, zgp_ref, h_ref, wz_ref, wzgp_ref, wh_ref, b_ref,
             wms_ref, wmd_ref, hid_ref, a_ref, b2_ref):
        hid = jnp.maximum(_dot(z_ref[...], wz_ref[...])
                          + _dot(zgp_ref[...], wzgp_ref[...])
                          + _dot(h_ref[...], wh_ref[...]) + b_ref[...], 0.0)
        hid_ref[...] = hid
        a_ref[...] = _dot(hid, wms_ref[...])
        b2_ref[...] = _dot(hid, wmd_ref[...])

    return _tc_call(body, n, _BN, [z, zgp, h, wz, wzgp, wh, b, wms, wmd],
                    [_f32((n, _LAT))] * 3)


def _ce_cat(ea, we, be, wmecat, bmcat):
    n = ea.shape[0]

    def body(ea_ref, we_ref, be_ref, w_ref, bcat_ref, c_ref):
        e = jnp.maximum(_dot(ea_ref[...], we_ref[...]) + be_ref[...], 0.0)
        c_ref[...] = _dot(e.astype(jnp.bfloat16),
                          w_ref[...].astype(jnp.bfloat16)) + bcat_ref[...]

    return _tc_call(body, n, _BE, [ea, we, be, wmecat, bmcat],
                    [_f32((n, 2 * _LAT))])[0]


def _upd_mid(hid, p0, p1, wuh, wua, bu, wms, wmd):
    n = hid.shape[0]

    def body(hid_ref, p0_ref, p1_ref, wuh_ref, wua_ref, bu_ref,
             wms_ref, wmd_ref, o_ref, a_ref, b_ref):
        agg = p0_ref[...] + p1_ref[...]
        nh = jnp.maximum(_dot(hid_ref[...], wuh_ref[...])
                         + _dot(agg, wua_ref[...]) + bu_ref[...], 0.0)
        o_ref[...] = nh
        a_ref[...] = _dot(nh, wms_ref[...])
        b_ref[...] = _dot(nh, wmd_ref[...])

    return _tc_call(body, n, _BN, [hid, p0, p1, wuh, wua, bu, wms, wmd],
                    [_f32((n, _LAT))] * 3)


def _upd_last(hid, p0, p1, wuh, wua, bu, wout, bout):
    n = hid.shape[0]
    dout = wout.shape[1]

    def body(hid_ref, p0_ref, p1_ref, wuh_ref, wua_ref, bu_ref,
             wo_ref, bo_ref, z_ref):
        agg = p0_ref[...] + p1_ref[...]
        nh = jnp.maximum(_dot(hid_ref[...], wuh_ref[...])
                         + _dot(agg, wua_ref[...]) + bu_ref[...], 0.0)
        z_ref[...] = _dot(nh, wo_ref[...]) + bo_ref[...]

    return _tc_call(body, n, _BN, [hid, p0, p1, wuh, wua, bu, wout, bout],
                    [_f32((n, dout))])[0]


def _vae_prior(zgp, w1, b1, w2, b2):
    n = zgp.shape[0]

    def body(z_ref, w1_ref, b1_ref, w2_ref, b2_ref, loc_ref, sc_ref):
        hdn = jnp.maximum(_dot(z_ref[...], w1_ref[...]) + b1_ref[...], 0.0)
        out = _dot(hdn, w2_ref[...]) + b2_ref[...]
        loc_ref[...] = out[:, :_LAT]
        sc_ref[...] = _softplus(out[:, _LAT:]) + 1e-4

    return _tc_call(body, n, _BN, [zgp, w1, b1, w2, b2],
                    [_f32((n, _LAT))] * 2)


def _vae_inf(zg, zgp, epsv, w1a, w1b, b1, w2, b2):
    n = zg.shape[0]

    def body(zg_ref, zgp_ref, e_ref, w1a_ref, w1b_ref, b1_ref, w2_ref, b2_ref,
             loc_ref, sc_ref, z_ref):
        hdn = jnp.maximum(_dot(zg_ref[...], w1a_ref[...])
                          + _dot(zgp_ref[...], w1b_ref[...]) + b1_ref[...], 0.0)
        out = _dot(hdn, w2_ref[...]) + b2_ref[...]
        loc = out[:, :_LAT]
        sc = _softplus(out[:, _LAT:]) + 1e-4
        loc_ref[...] = loc
        sc_ref[...] = sc
        z_ref[...] = loc + sc * e_ref[...]

    return _tc_call(body, n, _BN, [zg, zgp, epsv, w1a, w1b, b1, w2, b2],
                    [_f32((n, _LAT))] * 3)


_NC = 2
_NS = 16
_NW = _NC * _NS
_EPW = _E // _NW
_SUB = 50
_NCH = _EPW // _SUB
_RB = 624
_ZR = 48
_NB = 4


def _sc_edge_call(ce_off):
    mesh = plsc.VectorSubcoreMesh(core_axis_name="c", subcore_axis_name="s")
    scratch = [
        pltpu.VMEM((_NCH, _SUB), jnp.int32),
        pltpu.VMEM((_NCH, _SUB), jnp.int32),
        pltpu.VMEM((_NB, _SUB, _LAT), jnp.float32),
        pltpu.VMEM((_NB, _SUB, _LAT), jnp.float32),
        pltpu.VMEM((_ZR, _LAT), jnp.float32),
        pltpu.VMEM_SHARED((_N, _LAT), jnp.float32),
    ] + [pltpu.SemaphoreType.DMA] * (2 * _NB)
    has_ce = ce_off is not None
    if has_ce:
        scratch.insert(4, pltpu.VMEM((_NB, _SUB, _LAT), jnp.float32))
    else:
        scratch.insert(4, pltpu.VMEM((_LAT,), jnp.float32))

    def body(*refs):
        if has_ce:
            (a_hbm, b_hbm, ce_hbm, src_hbm, dst_hbm, out_hbm,
             sidx2, didx2, bufa, bufb, bufc, zb, agg) = refs[:13]
        else:
            (a_hbm, b_hbm, bias_hbm, src_hbm, dst_hbm, out_hbm,
             sidx2, didx2, bufa, bufb, bias_v, zb, agg) = refs[:13]
        fs = refs[13:13 + _NB]
        ss = refs[13 + _NB:13 + 2 * _NB]
        c = lax.axis_index("c")
        s = lax.axis_index("s")
        wid = s * _NC + c

        pltpu.sync_copy(src_hbm.at[wid], sidx2)
        pltpu.sync_copy(dst_hbm.at[wid], didx2)

        @pl.loop(0, _ZR)
        def _zero_rows(r):
            z = jnp.zeros((16,), jnp.float32)
            for kk in range(4):
                zb[r, pl.ds(kk * 16, 16)] = z

        for j in range(_RB // _ZR):
            pltpu.sync_copy(zb, agg.at[pl.ds(s * _RB + j * _ZR, _ZR), :])

        @pl.when(s == _NS - 1)
        def _zero_tail():
            pltpu.sync_copy(zb.at[pl.ds(0, _N - _NS * _RB), :],
                            agg.at[pl.ds(_NS * _RB, _N - _NS * _RB), :])
        if not has_ce:
            pltpu.sync_copy(bias_hbm, bias_v)
        plsc.subcore_barrier()

        def fire_fetch(ch, b):
            pltpu.make_async_copy(a_hbm.at[sidx2.at[ch]], bufa.at[b],
                                  fs[b]).start()
            pltpu.make_async_copy(b_hbm.at[didx2.at[ch]], bufb.at[b],
                                  fs[b]).start()
            if has_ce:
                base = wid * _EPW + ch * _SUB
                pltpu.make_async_copy(
                    ce_hbm.at[pl.ds(base, _SUB), pl.ds(ce_off, _LAT)],
                    bufc.at[b], fs[b]).start()

        def wait_fetch(ch, b):
            pltpu.make_async_copy(a_hbm.at[sidx2.at[ch]], bufa.at[b],
                                  fs[b]).wait()
            pltpu.make_async_copy(b_hbm.at[didx2.at[ch]], bufb.at[b],
                                  fs[b]).wait()
            if has_ce:
                base = wid * _EPW + ch * _SUB
                pltpu.make_async_copy(
                    ce_hbm.at[pl.ds(base, _SUB), pl.ds(ce_off, _LAT)],
                    bufc.at[b], fs[b]).wait()

        def wait_scatter(ch, b):
            pltpu.make_async_copy(bufa.at[b], agg.at[didx2.at[ch]],
                                  ss[b]).wait()

        for b in range(_NB - 1):
            fire_fetch(b, b)

        if not has_ce:
            bv = [bias_v[pl.ds(kk * 16, 16)] for kk in range(4)]

        @pl.loop(0, _NCH, step=_NB)
        def _outer(i0):
            for b in range(_NB):
                ch = i0 + b
                nf = ch + _NB - 1
                bf = (b + _NB - 1) % _NB

                def _refill():
                    wait_scatter(ch, bf)
                    fire_fetch(nf, bf)

                if b == 0:
                    @pl.when(i0 > 0)
                    def _():
                        _refill()

                    @pl.when(i0 == 0)
                    def _():
                        fire_fetch(nf, bf)
                else:
                    @pl.when(nf < _NCH)
                    def _():
                        _refill()

                wait_fetch(ch, b)

                if has_ce:
                    @plsc.parallel_loop(0, _SUB, unroll=4)
                    def _relu(r):
                        for kk in range(4):
                            sl = pl.ds(kk * 16, 16)
                            bufa[b, r, sl] = jnp.maximum(
                                bufa[b, r, sl] + bufb[b, r, sl]
                                + bufc[b, r, sl], 0.0)
                else:
                    @plsc.parallel_loop(0, _SUB, unroll=4)
                    def _relu(r):
                        for kk in range(4):
                            sl = pl.ds(kk * 16, 16)
                            bufa[b, r, sl] = jnp.maximum(
                                bufa[b, r, sl] + bufb[b, r, sl] + bv[kk], 0.0)

                pltpu.make_async_copy(bufa.at[b], agg.at[didx2.at[ch]],
                                      ss[b]).start(add=True)

        for b in range(_NB):
            wait_scatter(_NCH - _NB + b, b)

        plsc.subcore_barrier()
        for j in range(_RB // _ZR):
            pltpu.sync_copy(agg.at[pl.ds(s * _RB + j * _ZR, _ZR), :],
                            out_hbm.at[c, pl.ds(s * _RB + j * _ZR, _ZR), :])

        @pl.when(s == _NS - 1)
        def _write_tail():
            pltpu.sync_copy(agg.at[pl.ds(_NS * _RB, _N - _NS * _RB), :],
                            out_hbm.at[c, pl.ds(_NS * _RB, _N - _NS * _RB), :])

    return pl.kernel(
        body, mesh=mesh,
        out_type=jax.ShapeDtypeStruct((_NC, _N, _LAT), jnp.float32),
        scratch_types=scratch,
        compiler_params=pltpu.CompilerParams(use_tc_tiling_on_sc=False),
    )


def _edge_stage(a_nodes, b_nodes, ce_and_off, src, dst, dec_bias=None):
    src3 = src.reshape(_NW, _NCH, _SUB)
    dst3 = dst.reshape(_NW, _NCH, _SUB)
    if ce_and_off is not None:
        cec, off = ce_and_off
        parts = _sc_edge_call(off)(a_nodes, b_nodes, cec, src3, dst3)
    else:
        parts = _sc_edge_call(None)(a_nodes, b_nodes, dec_bias.reshape(_LAT),
                                    src3, dst3)
    return parts[0], parts[1]


def _r2(v):
    return v.reshape(1, -1)


def _gnn_encode(p, x, h, maskcol, ea, src, dst):
    l0, l1 = p["layers"]
    win = p["W_in"]
    dx = win.shape[0] - h.shape[1]
    hid, a0, b0 = _enc_in(x, h, maskcol, win[:dx], win[dx:], _r2(p["b_in"]),
                          l0["W_m"][:_LAT], l0["W_m"][_LAT:2 * _LAT])
    cec = _ce_cat(ea, p["W_e"], _r2(p["b_e"]),
                  jnp.concatenate([l0["W_m"][2 * _LAT:],
                                   l1["W_m"][2 * _LAT:]], axis=1),
                  jnp.concatenate([_r2(l0["b_m"]), _r2(l1["b_m"])], axis=1))
    q0, q1 = _edge_stage(a0, b0, (cec, 0), src, dst)
    hid, a1, b1 = _upd_mid(hid, q0, q1, l0["W_u"][:_LAT], l0["W_u"][_LAT:],
                           _r2(l0["b_u"]), l1["W_m"][:_LAT],
                           l1["W_m"][_LAT:2 * _LAT])
    q0, q1 = _edge_stage(a1, b1, (cec, _LAT), src, dst)
    return _upd_last(hid, q0, q1, l1["W_u"][:_LAT], l1["W_u"][_LAT:],
                     _r2(l1["b_u"]), p["W_out"], _r2(p["b_out"]))


def _gnn_decode(p, z, zgp, h, src, dst):
    l0, l1 = p["layers"]
    win = p["W_in"]
    e_const = jnp.maximum(p["b_e"], 0.0)
    cb0 = _r2(e_const @ l0["W_m"][2 * _LAT:] + l0["b_m"])
    cb1 = _r2(e_const @ l1["W_m"][2 * _LAT:] + l1["b_m"])
    hid, a0, b0 = _dec_in(z, zgp, h, win[:_LAT], win[_LAT:2 * _LAT],
                          win[2 * _LAT:], _r2(p["b_in"]),
                          l0["W_m"][:_LAT], l0["W_m"][_LAT:2 * _LAT])
    q0, q1 = _edge_stage(a0, b0, None, src, dst, dec_bias=cb0)
    hid, a1, b1 = _upd_mid(hid, q0, q1, l0["W_u"][:_LAT], l0["W_u"][_LAT:],
                           _r2(l0["b_u"]), l1["W_m"][:_LAT],
                           l1["W_m"][_LAT:2 * _LAT])
    q0, q1 = _edge_stage(a1, b1, None, src, dst, dec_bias=cb1)
    return _upd_last(hid, q0, q1, l1["W_u"][:_LAT], l1["W_u"][_LAT:],
                     _r2(l1["b_u"]), p["W_out"], _r2(p["b_out"]))


def kernel(x, h, edge_attr, edge_attr_partial, edge_index, partial_goal_mask,
           params, eps):
    src = edge_index[0]
    dst = edge_index[1]
    ones = jnp.ones((x.shape[0], 1), jnp.float32)
    maskcol = partial_goal_mask[:, None]

    z_goal = _gnn_encode(params["goal"], x, h, ones, edge_attr, src, dst)
    z_gp = _gnn_encode(params["partial"], x, h, maskcol, edge_attr_partial,
                       src, dst)

    pv = params["prior"]
    loc_p, scale_p = _vae_prior(z_gp, pv["W1"], _r2(pv["b1"]),
                                pv["W2"], _r2(pv["b2"]))
    iv = params["inf"]
    loc_q, scale_q, z = _vae_inf(z_goal, z_gp, eps,
                                 iv["W1"][:_LAT], iv["W1"][_LAT:], _r2(iv["b1"]),
                                 iv["W2"], _r2(iv["b2"]))

    mu = _gnn_decode(params["dec"], z, z_gp, h, src, dst)
    return mu, loc_q, scale_q, loc_p, scale_p
